# baseline (speedup 1.0000x reference)
import functools

import jax
import jax.numpy as jnp
from jax import lax
from jax.experimental import pallas as pl
from jax.experimental.pallas import tpu as pltpu
from jax.experimental.pallas import tpu_sc as plsc

B, N = 32, 4096
G, K = 256, 32
L = 16
NV = N // L
HB = 384
HLO = 112 * 16
CAND = 256
C = 8
BIG_I32 = 0x7FFFFFFF


def _splat(x):
    return jnp.full((L,), x)


def _bf16_round(v):
    bits = plsc.bitcast(v, jnp.int32)
    rnd = bits + 0x7FFF + jnp.bitwise_and(lax.shift_right_logical(bits, 16), 1)
    return plsc.bitcast(jnp.bitwise_and(rnd, -0x10000), jnp.float32)


def _merge32(Ak, Av, Bk, Bv, ck, cv):
    rk, rv = lax.rev(ck, (0,)), lax.rev(cv, (0,))
    m1 = Bk <= rk
    l1k = jnp.where(m1, Bk, rk)
    l1v = jnp.where(m1, Bv, rv)
    b1k, b1v = plsc.sort_key_val(l1k, l1v)
    rk2, rv2 = lax.rev(b1k, (0,)), lax.rev(b1v, (0,))
    m2 = Ak <= rk2
    lok = jnp.where(m2, Ak, rk2)
    lov = jnp.where(m2, Av, rv2)
    hik = jnp.where(m2, rk2, Ak)
    hiv = jnp.where(m2, rv2, Av)
    Ak, Av = plsc.sort_key_val(lok, lov)
    Bk, Bv = plsc.sort_key_val(hik, hiv)
    return Ak, Av, Bk, Bv


def _body(xyz_hbm, nb_hbm, ct_hbm,
          px, py, pz, xb, yb, zb, sx2, dmin, cidx,
          drow, hist, cand_d, cand_i, nb_v, ct_v):
    c = lax.axis_index("c")
    s = lax.axis_index("s")
    b = s * 2 + c
    pltpu.sync_copy(xyz_hbm.at[pl.ds(b * 3 * N, N)], px)
    pltpu.sync_copy(xyz_hbm.at[pl.ds(b * 3 * N + N, N)], py)
    pltpu.sync_copy(xyz_hbm.at[pl.ds(b * 3 * N + 2 * N, N)], pz)

    lanes = lax.iota(jnp.int32, L)
    ones = jnp.ones((L,), jnp.int32)
    inf_k = jnp.full((L,), jnp.inf, jnp.float32)
    zero_v = jnp.zeros((L,), jnp.int32)

    def init_pass(j, _):
        base = j * L
        dmin[pl.ds(base, L)] = jnp.full((L,), 1e10, jnp.float32)
        x = px[pl.ds(base, L)]
        y = py[pl.ds(base, L)]
        z = pz[pl.ds(base, L)]
        xb[pl.ds(base, L)] = _bf16_round(x)
        yb[pl.ds(base, L)] = _bf16_round(y)
        zb[pl.ds(base, L)] = _bf16_round(z)
        sx2[pl.ds(base, L)] = (x * x + y * y) + z * z
        return 0
    lax.fori_loop(0, NV, init_pass, 0)

    def clr_hist(j, _):
        hist[pl.ds(j * L, L)] = jnp.zeros((L,), jnp.int32)
        return 0
    lax.fori_loop(0, C * HB // L, clr_hist, 0)

    def fps_iter(i, far):
        far_v = _splat(far)
        cx = plsc.load_gather(px, [far_v])
        cy = plsc.load_gather(py, [far_v])
        cz = plsc.load_gather(pz, [far_v])
        cvals = jnp.where(lanes == 0, cx, jnp.where(lanes == 1, cy, cz))
        plsc.store_scatter(ct_v, [3 * _splat(i) + lanes], cvals, mask=lanes < 3)
        plsc.store_scatter(cidx, [_splat(i)], far_v, mask=lanes < 1)

        def pass_a(j, carry):
            out = []
            for u in range(4):
                rmax, ridx = carry[2 * u], carry[2 * u + 1]
                base = (4 * j + u) * L
                x = px[pl.ds(base, L)]
                y = py[pl.ds(base, L)]
                z = pz[pl.ds(base, L)]
                dx = x - cx
                dy = y - cy
                dz = z - cz
                d = (dx * dx + dy * dy) + dz * dz
                dm = jnp.minimum(dmin[pl.ds(base, L)], d)
                dmin[pl.ds(base, L)] = dm
                upd = dm > rmax
                out.append(jnp.where(upd, dm, rmax))
                out.append(jnp.where(upd, base + lanes, ridx))
            return tuple(out)

        init = (jnp.full((L,), -1.0, jnp.float32),
                jnp.zeros((L,), jnp.int32)) * 4
        carry = plsc.parallel_loop(0, NV // 4, carry=init)(pass_a)

        def comb(va, ia, vb, ib):
            selb = jnp.logical_or(
                vb > va, jnp.logical_and(vb == va, ib < ia))
            return jnp.where(selb, vb, va), jnp.where(selb, ib, ia)
        r0, i0 = comb(carry[0], carry[1], carry[2], carry[3])
        r1, i1 = comb(carry[4], carry[5], carry[6], carry[7])
        rmax, ridx = comb(r0, i0, r1, i1)
        gmax = jnp.max(rmax)
        return jnp.min(jnp.where(rmax == gmax, ridx, BIG_I32))

    lax.fori_loop(0, G, fps_iter, jnp.int32(0))

    def knn_group(g, _):
        row0 = g * C
        cxb2s, cyb2s, czb2s, sc2s = [], [], [], []
        for t in range(C):
            fv = plsc.load_gather(cidx, [_splat(row0 + t)])
            cx = plsc.load_gather(px, [fv])
            cy = plsc.load_gather(py, [fv])
            cz = plsc.load_gather(pz, [fv])
            cxb = _bf16_round(cx)
            cyb = _bf16_round(cy)
            czb = _bf16_round(cz)
            cxb2s.append(cxb + cxb)
            cyb2s.append(cyb + cyb)
            czb2s.append(czb + czb)
            sc2s.append((cx * cx + cy * cy) + cz * cz)

        def sweep(j, _):
            for u in range(2):
                base = (2 * j + u) * L
                xbv = xb[pl.ds(base, L)]
                ybv = yb[pl.ds(base, L)]
                zbv = zb[pl.ds(base, L)]
                sxv = sx2[pl.ds(base, L)]
                for t in range(C):
                    dot2 = (cxb2s[t] * xbv + cyb2s[t] * ybv) + czb2s[t] * zbv
                    ds = (sc2s[t] - dot2) + sxv
                    drow[pl.ds(t * N + base, L)] = ds
                    bits = plsc.bitcast(ds, jnp.int32)
                    bk = jnp.clip(lax.shift_right_arithmetic(bits, 19) - HLO,
                                  0, HB - 1)
                    plsc.addupdate_scatter(hist, [t * HB + bk], ones)
            return j
        plsc.parallel_loop(0, NV // 2, carry=jnp.int32(0))(
            lambda j, _: sweep(j, _))

        for t in range(C):
            def scan_hist(j, carry):
                cum, bstar, done = carry
                h = hist[pl.ds(t * HB + j * L, L)]
                hist[pl.ds(t * HB + j * L, L)] = zero_v
                csum = plsc.cumsum(h) + cum
                lane = jnp.min(jnp.where(csum >= K, lanes, jnp.int32(L)))
                found = lane < L
                take = jnp.logical_and(found, jnp.logical_not(done))
                bstar = jnp.where(take, j * L + lane, bstar)
                done = jnp.logical_or(done, found)
                cum = cum + jnp.sum(h)
                return cum, bstar, done
            _, bstar, _ = lax.fori_loop(
                0, HB // L, scan_hist,
                (jnp.int32(0), jnp.int32(HB - 1), jnp.bool_(False)))
            tf_bits = lax.shift_left(bstar + (HLO + 1), 19)
            tf = plsc.bitcast(jnp.full((L,), tf_bits), jnp.float32)

            def clr_cand(j, _):
                cand_d[pl.ds(j * L, L)] = inf_k
                return 0
            lax.fori_loop(0, CAND // L, clr_cand, 0)

            def pass_b(j, basel):
                for u in range(4):
                    base = (4 * j + u) * L
                    d = drow[pl.ds(t * N + base, L)]
                    m = d < tf
                    pos = jnp.minimum(lax.shift_left(basel, 4) + lanes,
                                      CAND - 1)
                    plsc.store_scatter(cand_d, [pos], d, mask=m)
                    plsc.store_scatter(cand_i, [pos], base + lanes, mask=m)
                    basel = basel + m.astype(jnp.int32)
                return basel
            basel = plsc.parallel_loop(
                0, NV // 4, carry=jnp.zeros((L,), jnp.int32))(pass_b)

            def merge_step(u, carry):
                ak, av, bk2, bv2 = carry
                ck = cand_d[pl.ds(u * L, L)]
                cv = cand_i[pl.ds(u * L, L)]
                ck, cv = plsc.sort_key_val(ck, cv)
                return _merge32(ak, av, bk2, bv2, ck, cv)
            nvm = jnp.minimum(jnp.max(basel), CAND // L)
            Ak, Av, Bk, Bv = lax.fori_loop(
                0, nvm, merge_step, (inf_k, zero_v, inf_k, zero_v))

            fv = plsc.load_gather(cidx, [_splat(row0 + t)])
            cx = plsc.load_gather(px, [fv])
            cy = plsc.load_gather(py, [fv])
            cz = plsc.load_gather(pz, [fv])
            for vals, koff in ((Av, 0), (Bv, L)):
                xg = plsc.load_gather(px, [vals]) - cx
                yg = plsc.load_gather(py, [vals]) - cy
                zg = plsc.load_gather(pz, [vals]) - cz
                pos = 3 * (K * _splat(row0 + t) + koff + lanes)
                plsc.store_scatter(nb_v, [pos], xg)
                plsc.store_scatter(nb_v, [pos + 1], yg)
                plsc.store_scatter(nb_v, [pos + 2], zg)
        return 0

    lax.fori_loop(0, G // C, knn_group, 0)

    pltpu.sync_copy(nb_v, nb_hbm.at[pl.ds(b * (G * K * 3), G * K * 3)])
    pltpu.sync_copy(ct_v, ct_hbm.at[pl.ds(b * (G * 3), G * 3)])


@functools.partial(
    pl.kernel,
    out_type=(jax.ShapeDtypeStruct((B * G * K * 3,), jnp.float32),
              jax.ShapeDtypeStruct((B * G * 3,), jnp.float32)),
    mesh=plsc.VectorSubcoreMesh(core_axis_name="c", subcore_axis_name="s"),
    compiler_params=pltpu.CompilerParams(needs_layout_passes=False),
    scratch_types=[
        pltpu.VMEM((N,), jnp.float32),
        pltpu.VMEM((N,), jnp.float32),
        pltpu.VMEM((N,), jnp.float32),
        pltpu.VMEM((N,), jnp.float32),
        pltpu.VMEM((N,), jnp.float32),
        pltpu.VMEM((N,), jnp.float32),
        pltpu.VMEM((N,), jnp.float32),
        pltpu.VMEM((N,), jnp.float32),
        pltpu.VMEM((G,), jnp.int32),
        pltpu.VMEM((C * N,), jnp.float32),
        pltpu.VMEM((C * HB,), jnp.int32),
        pltpu.VMEM((CAND,), jnp.float32),
        pltpu.VMEM((CAND,), jnp.int32),
        pltpu.VMEM((G * K * 3,), jnp.float32),
        pltpu.VMEM((G * 3,), jnp.float32),
    ],
)
def _group_kernel(xyz_hbm, nb_hbm, ct_hbm, *scratch):
    _body(xyz_hbm, nb_hbm, ct_hbm, *scratch)


def kernel(xyz):
    xyz_flat = jnp.transpose(xyz, (0, 2, 1)).reshape(-1)
    nb_flat, ct_flat = _group_kernel(xyz_flat)
    return nb_flat.reshape(B, G, K, 3), ct_flat.reshape(B, G, 3)


# speedup vs baseline: 1.0293x; 1.0293x over previous
import functools

import jax
import jax.numpy as jnp
from jax import lax
from jax.experimental import pallas as pl
from jax.experimental.pallas import tpu as pltpu
from jax.experimental.pallas import tpu_sc as plsc

B, N = 32, 4096
G, K = 256, 32
L = 16
NV = N // L
HB = 384
HLO = 112 * 16
CAND = 256
C = 2
BIG_I32 = 0x7FFFFFFF


def _splat(x):
    return jnp.full((L,), x)


def _bf16_round(v):
    bits = plsc.bitcast(v, jnp.int32)
    rnd = bits + 0x7FFF + jnp.bitwise_and(lax.shift_right_logical(bits, 16), 1)
    return plsc.bitcast(jnp.bitwise_and(rnd, -0x10000), jnp.float32)


def _merge32(Ak, Av, Bk, Bv, ck, cv):
    rk, rv = lax.rev(ck, (0,)), lax.rev(cv, (0,))
    m1 = Bk <= rk
    l1k = jnp.where(m1, Bk, rk)
    l1v = jnp.where(m1, Bv, rv)
    b1k, b1v = plsc.sort_key_val(l1k, l1v)
    rk2, rv2 = lax.rev(b1k, (0,)), lax.rev(b1v, (0,))
    m2 = Ak <= rk2
    lok = jnp.where(m2, Ak, rk2)
    lov = jnp.where(m2, Av, rv2)
    hik = jnp.where(m2, rk2, Ak)
    hiv = jnp.where(m2, rv2, Av)
    Ak, Av = plsc.sort_key_val(lok, lov)
    Bk, Bv = plsc.sort_key_val(hik, hiv)
    return Ak, Av, Bk, Bv


def _body(xyz_hbm, nb_hbm, ct_hbm,
          px, py, pz, xb, yb, zb, sx2, dmin, cidx,
          drow, hist, cand_d, cand_i, nb_v, ct_v):
    c = lax.axis_index("c")
    s = lax.axis_index("s")
    b = s * 2 + c
    pltpu.sync_copy(xyz_hbm.at[pl.ds(b * 3 * N, N)], px)
    pltpu.sync_copy(xyz_hbm.at[pl.ds(b * 3 * N + N, N)], py)
    pltpu.sync_copy(xyz_hbm.at[pl.ds(b * 3 * N + 2 * N, N)], pz)

    lanes = lax.iota(jnp.int32, L)
    ones = jnp.ones((L,), jnp.int32)
    inf_k = jnp.full((L,), jnp.inf, jnp.float32)
    zero_v = jnp.zeros((L,), jnp.int32)

    def init_pass(j, _):
        base = j * L
        dmin[pl.ds(base, L)] = jnp.full((L,), 1e10, jnp.float32)
        x = px[pl.ds(base, L)]
        y = py[pl.ds(base, L)]
        z = pz[pl.ds(base, L)]
        xb[pl.ds(base, L)] = _bf16_round(x)
        yb[pl.ds(base, L)] = _bf16_round(y)
        zb[pl.ds(base, L)] = _bf16_round(z)
        sx2[pl.ds(base, L)] = (x * x + y * y) + z * z
        return 0
    lax.fori_loop(0, NV, init_pass, 0)

    def clr_hist(j, _):
        hist[pl.ds(j * L, L)] = jnp.zeros((L,), jnp.int32)
        return 0
    lax.fori_loop(0, C * HB // L, clr_hist, 0)

    def fps_iter(i, far):
        far_v = _splat(far)
        cx = plsc.load_gather(px, [far_v])
        cy = plsc.load_gather(py, [far_v])
        cz = plsc.load_gather(pz, [far_v])
        cvals = jnp.where(lanes == 0, cx, jnp.where(lanes == 1, cy, cz))
        plsc.store_scatter(ct_v, [3 * _splat(i) + lanes], cvals, mask=lanes < 3)
        plsc.store_scatter(cidx, [_splat(i)], far_v, mask=lanes < 1)

        def pass_a(j, carry):
            out = []
            for u in range(4):
                rmax, ridx = carry[2 * u], carry[2 * u + 1]
                base = (4 * j + u) * L
                x = px[pl.ds(base, L)]
                y = py[pl.ds(base, L)]
                z = pz[pl.ds(base, L)]
                dx = x - cx
                dy = y - cy
                dz = z - cz
                d = (dx * dx + dy * dy) + dz * dz
                dm = jnp.minimum(dmin[pl.ds(base, L)], d)
                dmin[pl.ds(base, L)] = dm
                upd = dm > rmax
                out.append(jnp.where(upd, dm, rmax))
                out.append(jnp.where(upd, base + lanes, ridx))
            return tuple(out)

        init = (jnp.full((L,), -1.0, jnp.float32),
                jnp.zeros((L,), jnp.int32)) * 4
        carry = plsc.parallel_loop(0, NV // 4, carry=init)(pass_a)

        def comb(va, ia, vb, ib):
            selb = jnp.logical_or(
                vb > va, jnp.logical_and(vb == va, ib < ia))
            return jnp.where(selb, vb, va), jnp.where(selb, ib, ia)
        r0, i0 = comb(carry[0], carry[1], carry[2], carry[3])
        r1, i1 = comb(carry[4], carry[5], carry[6], carry[7])
        rmax, ridx = comb(r0, i0, r1, i1)
        gmax = jnp.max(rmax)
        return jnp.min(jnp.where(rmax == gmax, ridx, BIG_I32))

    lax.fori_loop(0, G, fps_iter, jnp.int32(0))

    def knn_group(g, _):
        row0 = g * C
        cxb2s, cyb2s, czb2s, sc2s = [], [], [], []
        for t in range(C):
            fv = plsc.load_gather(cidx, [_splat(row0 + t)])
            cx = plsc.load_gather(px, [fv])
            cy = plsc.load_gather(py, [fv])
            cz = plsc.load_gather(pz, [fv])
            cxb = _bf16_round(cx)
            cyb = _bf16_round(cy)
            czb = _bf16_round(cz)
            cxb2s.append(cxb + cxb)
            cyb2s.append(cyb + cyb)
            czb2s.append(czb + czb)
            sc2s.append((cx * cx + cy * cy) + cz * cz)

        def sweep(j, _):
            for u in range(2):
                base = (2 * j + u) * L
                xbv = xb[pl.ds(base, L)]
                ybv = yb[pl.ds(base, L)]
                zbv = zb[pl.ds(base, L)]
                sxv = sx2[pl.ds(base, L)]
                for t in range(C):
                    dot2 = (cxb2s[t] * xbv + cyb2s[t] * ybv) + czb2s[t] * zbv
                    ds = (sc2s[t] - dot2) + sxv
                    drow[pl.ds(t * N + base, L)] = ds
                    bits = plsc.bitcast(ds, jnp.int32)
                    bk = jnp.clip(lax.shift_right_arithmetic(bits, 19) - HLO,
                                  0, HB - 1)
                    plsc.addupdate_scatter(hist, [t * HB + bk], ones)
            return j
        plsc.parallel_loop(0, NV // 2, carry=jnp.int32(0))(
            lambda j, _: sweep(j, _))

        for t in range(C):
            def scan_hist(j, carry):
                cum, bstar, done = carry
                h = hist[pl.ds(t * HB + j * L, L)]
                hist[pl.ds(t * HB + j * L, L)] = zero_v
                csum = plsc.cumsum(h) + cum
                lane = jnp.min(jnp.where(csum >= K, lanes, jnp.int32(L)))
                found = lane < L
                take = jnp.logical_and(found, jnp.logical_not(done))
                bstar = jnp.where(take, j * L + lane, bstar)
                done = jnp.logical_or(done, found)
                cum = cum + jnp.sum(h)
                return cum, bstar, done
            _, bstar, _ = lax.fori_loop(
                0, HB // L, scan_hist,
                (jnp.int32(0), jnp.int32(HB - 1), jnp.bool_(False)))
            tf_bits = lax.shift_left(bstar + (HLO + 1), 19)
            tf = plsc.bitcast(jnp.full((L,), tf_bits), jnp.float32)

            def clr_cand(j, _):
                cand_d[pl.ds(j * L, L)] = inf_k
                return 0
            lax.fori_loop(0, CAND // L, clr_cand, 0)

            def pass_b(j, basel):
                for u in range(4):
                    base = (4 * j + u) * L
                    d = drow[pl.ds(t * N + base, L)]
                    m = d < tf
                    pos = jnp.minimum(lax.shift_left(basel, 4) + lanes,
                                      CAND - 1)
                    plsc.store_scatter(cand_d, [pos], d, mask=m)
                    plsc.store_scatter(cand_i, [pos], base + lanes, mask=m)
                    basel = basel + m.astype(jnp.int32)
                return basel
            basel = plsc.parallel_loop(
                0, NV // 4, carry=jnp.zeros((L,), jnp.int32))(pass_b)

            def merge_step(u, carry):
                ak, av, bk2, bv2 = carry
                ck = cand_d[pl.ds(u * L, L)]
                cv = cand_i[pl.ds(u * L, L)]
                ck, cv = plsc.sort_key_val(ck, cv)
                return _merge32(ak, av, bk2, bv2, ck, cv)
            nvm = jnp.minimum(jnp.max(basel), CAND // L)
            Ak, Av, Bk, Bv = lax.fori_loop(
                0, nvm, merge_step, (inf_k, zero_v, inf_k, zero_v))

            fv = plsc.load_gather(cidx, [_splat(row0 + t)])
            cx = plsc.load_gather(px, [fv])
            cy = plsc.load_gather(py, [fv])
            cz = plsc.load_gather(pz, [fv])
            for vals, koff in ((Av, 0), (Bv, L)):
                xg = plsc.load_gather(px, [vals]) - cx
                yg = plsc.load_gather(py, [vals]) - cy
                zg = plsc.load_gather(pz, [vals]) - cz
                pos = 3 * (K * _splat(row0 + t) + koff + lanes)
                plsc.store_scatter(nb_v, [pos], xg)
                plsc.store_scatter(nb_v, [pos + 1], yg)
                plsc.store_scatter(nb_v, [pos + 2], zg)
        return 0

    lax.fori_loop(0, G // C, knn_group, 0)

    pltpu.sync_copy(nb_v, nb_hbm.at[pl.ds(b * (G * K * 3), G * K * 3)])
    pltpu.sync_copy(ct_v, ct_hbm.at[pl.ds(b * (G * 3), G * 3)])


@functools.partial(
    pl.kernel,
    out_type=(jax.ShapeDtypeStruct((B * G * K * 3,), jnp.float32),
              jax.ShapeDtypeStruct((B * G * 3,), jnp.float32)),
    mesh=plsc.VectorSubcoreMesh(core_axis_name="c", subcore_axis_name="s"),
    compiler_params=pltpu.CompilerParams(needs_layout_passes=False),
    scratch_types=[
        pltpu.VMEM((N,), jnp.float32),
        pltpu.VMEM((N,), jnp.float32),
        pltpu.VMEM((N,), jnp.float32),
        pltpu.VMEM((N,), jnp.float32),
        pltpu.VMEM((N,), jnp.float32),
        pltpu.VMEM((N,), jnp.float32),
        pltpu.VMEM((N,), jnp.float32),
        pltpu.VMEM((N,), jnp.float32),
        pltpu.VMEM((G,), jnp.int32),
        pltpu.VMEM((C * N,), jnp.float32),
        pltpu.VMEM((C * HB,), jnp.int32),
        pltpu.VMEM((CAND,), jnp.float32),
        pltpu.VMEM((CAND,), jnp.int32),
        pltpu.VMEM((G * K * 3,), jnp.float32),
        pltpu.VMEM((G * 3,), jnp.float32),
    ],
)
def _group_kernel(xyz_hbm, nb_hbm, ct_hbm, *scratch):
    _body(xyz_hbm, nb_hbm, ct_hbm, *scratch)


def kernel(xyz):
    xyz_flat = jnp.transpose(xyz, (0, 2, 1)).reshape(-1)
    nb_flat, ct_flat = _group_kernel(xyz_flat)
    return nb_flat.reshape(B, G, K, 3), ct_flat.reshape(B, G, 3)


# speedup vs baseline: 1.0353x; 1.0059x over previous
import functools

import jax
import jax.numpy as jnp
from jax import lax
from jax.experimental import pallas as pl
from jax.experimental.pallas import tpu as pltpu
from jax.experimental.pallas import tpu_sc as plsc

B, N = 32, 4096
G, K = 256, 32
L = 16
NV = N // L
HB = 384
HLO = 112 * 16
CAND = 256
C = 4
BIG_I32 = 0x7FFFFFFF


def _splat(x):
    return jnp.full((L,), x)


def _bf16_round(v):
    bits = plsc.bitcast(v, jnp.int32)
    rnd = bits + 0x7FFF + jnp.bitwise_and(lax.shift_right_logical(bits, 16), 1)
    return plsc.bitcast(jnp.bitwise_and(rnd, -0x10000), jnp.float32)


def _merge32(Ak, Av, Bk, Bv, ck, cv):
    rk, rv = lax.rev(ck, (0,)), lax.rev(cv, (0,))
    m1 = Bk <= rk
    l1k = jnp.where(m1, Bk, rk)
    l1v = jnp.where(m1, Bv, rv)
    b1k, b1v = plsc.sort_key_val(l1k, l1v)
    rk2, rv2 = lax.rev(b1k, (0,)), lax.rev(b1v, (0,))
    m2 = Ak <= rk2
    lok = jnp.where(m2, Ak, rk2)
    lov = jnp.where(m2, Av, rv2)
    hik = jnp.where(m2, rk2, Ak)
    hiv = jnp.where(m2, rv2, Av)
    Ak, Av = plsc.sort_key_val(lok, lov)
    Bk, Bv = plsc.sort_key_val(hik, hiv)
    return Ak, Av, Bk, Bv


def _body(xyz_hbm, nb_hbm, ct_hbm,
          px, py, pz, xb, yb, zb, sx2, dmin, cidx,
          drow, hist, cand_d, cand_i, nb_v, ct_v):
    c = lax.axis_index("c")
    s = lax.axis_index("s")
    b = s * 2 + c
    pltpu.sync_copy(xyz_hbm.at[pl.ds(b * 3 * N, N)], px)
    pltpu.sync_copy(xyz_hbm.at[pl.ds(b * 3 * N + N, N)], py)
    pltpu.sync_copy(xyz_hbm.at[pl.ds(b * 3 * N + 2 * N, N)], pz)

    lanes = lax.iota(jnp.int32, L)
    ones = jnp.ones((L,), jnp.int32)
    inf_k = jnp.full((L,), jnp.inf, jnp.float32)
    zero_v = jnp.zeros((L,), jnp.int32)

    def init_pass(j, _):
        base = j * L
        dmin[pl.ds(base, L)] = jnp.full((L,), 1e10, jnp.float32)
        x = px[pl.ds(base, L)]
        y = py[pl.ds(base, L)]
        z = pz[pl.ds(base, L)]
        xb[pl.ds(base, L)] = _bf16_round(x)
        yb[pl.ds(base, L)] = _bf16_round(y)
        zb[pl.ds(base, L)] = _bf16_round(z)
        sx2[pl.ds(base, L)] = (x * x + y * y) + z * z
        return 0
    lax.fori_loop(0, NV, init_pass, 0)

    def clr_hist(j, _):
        hist[pl.ds(j * L, L)] = jnp.zeros((L,), jnp.int32)
        return 0
    lax.fori_loop(0, C * HB // L, clr_hist, 0)

    def fps_iter(i, far):
        far_v = _splat(far)
        cx = plsc.load_gather(px, [far_v])
        cy = plsc.load_gather(py, [far_v])
        cz = plsc.load_gather(pz, [far_v])
        cvals = jnp.where(lanes == 0, cx, jnp.where(lanes == 1, cy, cz))
        plsc.store_scatter(ct_v, [3 * _splat(i) + lanes], cvals, mask=lanes < 3)
        plsc.store_scatter(cidx, [_splat(i)], far_v, mask=lanes < 1)

        def pass_a(j, carry):
            out = []
            for u in range(4):
                rmax, ridx = carry[2 * u], carry[2 * u + 1]
                base = (4 * j + u) * L
                x = px[pl.ds(base, L)]
                y = py[pl.ds(base, L)]
                z = pz[pl.ds(base, L)]
                dx = x - cx
                dy = y - cy
                dz = z - cz
                d = (dx * dx + dy * dy) + dz * dz
                dm = jnp.minimum(dmin[pl.ds(base, L)], d)
                dmin[pl.ds(base, L)] = dm
                upd = dm > rmax
                out.append(jnp.where(upd, dm, rmax))
                out.append(jnp.where(upd, base + lanes, ridx))
            return tuple(out)

        init = (jnp.full((L,), -1.0, jnp.float32),
                jnp.zeros((L,), jnp.int32)) * 4
        carry = plsc.parallel_loop(0, NV // 4, carry=init)(pass_a)

        def comb(va, ia, vb, ib):
            selb = jnp.logical_or(
                vb > va, jnp.logical_and(vb == va, ib < ia))
            return jnp.where(selb, vb, va), jnp.where(selb, ib, ia)
        r0, i0 = comb(carry[0], carry[1], carry[2], carry[3])
        r1, i1 = comb(carry[4], carry[5], carry[6], carry[7])
        rmax, ridx = comb(r0, i0, r1, i1)
        gmax = jnp.max(rmax)
        return jnp.min(jnp.where(rmax == gmax, ridx, BIG_I32))

    lax.fori_loop(0, G, fps_iter, jnp.int32(0))

    def knn_group(g, _):
        row0 = g * C
        cxb2s, cyb2s, czb2s, sc2s = [], [], [], []
        for t in range(C):
            fv = plsc.load_gather(cidx, [_splat(row0 + t)])
            cx = plsc.load_gather(px, [fv])
            cy = plsc.load_gather(py, [fv])
            cz = plsc.load_gather(pz, [fv])
            cxb = _bf16_round(cx)
            cyb = _bf16_round(cy)
            czb = _bf16_round(cz)
            cxb2s.append(cxb + cxb)
            cyb2s.append(cyb + cyb)
            czb2s.append(czb + czb)
            sc2s.append((cx * cx + cy * cy) + cz * cz)

        def sweep(j, _):
            for u in range(2):
                base = (2 * j + u) * L
                xbv = xb[pl.ds(base, L)]
                ybv = yb[pl.ds(base, L)]
                zbv = zb[pl.ds(base, L)]
                sxv = sx2[pl.ds(base, L)]
                for t in range(C):
                    dot2 = (cxb2s[t] * xbv + cyb2s[t] * ybv) + czb2s[t] * zbv
                    ds = (sc2s[t] - dot2) + sxv
                    drow[pl.ds(t * N + base, L)] = ds
                    bits = plsc.bitcast(ds, jnp.int32)
                    bk = jnp.clip(lax.shift_right_arithmetic(bits, 19) - HLO,
                                  0, HB - 1)
                    plsc.addupdate_scatter(hist, [t * HB + bk], ones)
            return j
        plsc.parallel_loop(0, NV // 2, carry=jnp.int32(0))(
            lambda j, _: sweep(j, _))

        for t in range(C):
            def scan_hist(j, carry):
                cum, bstar, done = carry
                h = hist[pl.ds(t * HB + j * L, L)]
                hist[pl.ds(t * HB + j * L, L)] = zero_v
                csum = plsc.cumsum(h) + cum
                lane = jnp.min(jnp.where(csum >= K, lanes, jnp.int32(L)))
                found = lane < L
                take = jnp.logical_and(found, jnp.logical_not(done))
                bstar = jnp.where(take, j * L + lane, bstar)
                done = jnp.logical_or(done, found)
                cum = cum + jnp.sum(h)
                return cum, bstar, done
            _, bstar, _ = lax.fori_loop(
                0, HB // L, scan_hist,
                (jnp.int32(0), jnp.int32(HB - 1), jnp.bool_(False)))
            tf_bits = lax.shift_left(bstar + (HLO + 1), 19)
            tf = plsc.bitcast(jnp.full((L,), tf_bits), jnp.float32)

            def clr_cand(j, _):
                cand_d[pl.ds(j * L, L)] = inf_k
                return 0
            lax.fori_loop(0, CAND // L, clr_cand, 0)

            def pass_b(j, basel):
                for u in range(4):
                    base = (4 * j + u) * L
                    d = drow[pl.ds(t * N + base, L)]
                    m = d < tf
                    pos = jnp.minimum(lax.shift_left(basel, 4) + lanes,
                                      CAND - 1)
                    plsc.store_scatter(cand_d, [pos], d, mask=m)
                    plsc.store_scatter(cand_i, [pos], base + lanes, mask=m)
                    basel = basel + m.astype(jnp.int32)
                return basel
            basel = plsc.parallel_loop(
                0, NV // 4, carry=jnp.zeros((L,), jnp.int32))(pass_b)

            def merge_step(u, carry):
                ak, av, bk2, bv2 = carry
                ck = cand_d[pl.ds(u * L, L)]
                cv = cand_i[pl.ds(u * L, L)]
                ck, cv = plsc.sort_key_val(ck, cv)
                return _merge32(ak, av, bk2, bv2, ck, cv)
            nvm = jnp.minimum(jnp.max(basel), CAND // L)
            Ak, Av, Bk, Bv = lax.fori_loop(
                0, nvm, merge_step, (inf_k, zero_v, inf_k, zero_v))

            fv = plsc.load_gather(cidx, [_splat(row0 + t)])
            cx = plsc.load_gather(px, [fv])
            cy = plsc.load_gather(py, [fv])
            cz = plsc.load_gather(pz, [fv])
            for vals, koff in ((Av, 0), (Bv, L)):
                xg = plsc.load_gather(px, [vals]) - cx
                yg = plsc.load_gather(py, [vals]) - cy
                zg = plsc.load_gather(pz, [vals]) - cz
                pos = 3 * (K * _splat(row0 + t) + koff + lanes)
                plsc.store_scatter(nb_v, [pos], xg)
                plsc.store_scatter(nb_v, [pos + 1], yg)
                plsc.store_scatter(nb_v, [pos + 2], zg)
        return 0

    lax.fori_loop(0, G // C, knn_group, 0)

    pltpu.sync_copy(nb_v, nb_hbm.at[pl.ds(b * (G * K * 3), G * K * 3)])
    pltpu.sync_copy(ct_v, ct_hbm.at[pl.ds(b * (G * 3), G * 3)])


@functools.partial(
    pl.kernel,
    out_type=(jax.ShapeDtypeStruct((B * G * K * 3,), jnp.float32),
              jax.ShapeDtypeStruct((B * G * 3,), jnp.float32)),
    mesh=plsc.VectorSubcoreMesh(core_axis_name="c", subcore_axis_name="s"),
    compiler_params=pltpu.CompilerParams(needs_layout_passes=False),
    scratch_types=[
        pltpu.VMEM((N,), jnp.float32),
        pltpu.VMEM((N,), jnp.float32),
        pltpu.VMEM((N,), jnp.float32),
        pltpu.VMEM((N,), jnp.float32),
        pltpu.VMEM((N,), jnp.float32),
        pltpu.VMEM((N,), jnp.float32),
        pltpu.VMEM((N,), jnp.float32),
        pltpu.VMEM((N,), jnp.float32),
        pltpu.VMEM((G,), jnp.int32),
        pltpu.VMEM((C * N,), jnp.float32),
        pltpu.VMEM((C * HB,), jnp.int32),
        pltpu.VMEM((CAND,), jnp.float32),
        pltpu.VMEM((CAND,), jnp.int32),
        pltpu.VMEM((G * K * 3,), jnp.float32),
        pltpu.VMEM((G * 3,), jnp.float32),
    ],
)
def _group_kernel(xyz_hbm, nb_hbm, ct_hbm, *scratch):
    _body(xyz_hbm, nb_hbm, ct_hbm, *scratch)


def kernel(xyz):
    xyz_flat = jnp.transpose(xyz, (0, 2, 1)).reshape(-1)
    nb_flat, ct_flat = _group_kernel(xyz_flat)
    return nb_flat.reshape(B, G, K, 3), ct_flat.reshape(B, G, 3)
